# baseline (speedup 1.0000x reference)
import jax
import jax.numpy as jnp
from jax import lax
from jax.experimental import pallas as pl
from jax.experimental.pallas import tpu as pltpu

_K = 128
_STAGES = 10


def _dfc_body(x_ref, p0_ref, cluster_ref, proto_ref, sim_ref):
    xc = x_ref[0]
    xt = jnp.transpose(xc)
    p0 = p0_ref[0]
    n = xt.shape[0]

    def stage(_, carry):
        p, _ = carry
        sim = jnp.dot(xt, p, preferred_element_type=jnp.float32)
        m = jnp.max(sim, axis=-1, keepdims=True)
        e = jnp.exp(sim - m)
        s = e / jnp.sum(e, axis=-1, keepdims=True)
        w = s / jnp.sum(s, axis=0, keepdims=True)
        pn = p * 0.5 + jnp.dot(xc, w, preferred_element_type=jnp.float32) * 0.5
        return pn, s

    p, s = lax.fori_loop(
        0, _STAGES, stage,
        (p0, jnp.zeros((n, _K), jnp.float32)))

    sim_ref[0] = s
    proto_ref[0] = jnp.transpose(p)
    m = jnp.max(s, axis=-1, keepdims=True)
    idx = lax.broadcasted_iota(jnp.int32, (n, _K), 1)
    cluster_ref[0, 0] = jnp.min(jnp.where(s == m, idx, _K), axis=-1)


def kernel(x):
    b, c, n = x.shape
    p0 = x[:, :, :: n // _K]

    cluster3, proto, sim = pl.pallas_call(
        _dfc_body,
        grid=(b,),
        in_specs=[
            pl.BlockSpec((1, c, n), lambda i: (i, 0, 0)),
            pl.BlockSpec((1, c, _K), lambda i: (i, 0, 0)),
        ],
        out_specs=[
            pl.BlockSpec((1, 1, n), lambda i: (i, 0, 0)),
            pl.BlockSpec((1, _K, c), lambda i: (i, 0, 0)),
            pl.BlockSpec((1, n, _K), lambda i: (i, 0, 0)),
        ],
        out_shape=[
            jax.ShapeDtypeStruct((b, 1, n), jnp.int32),
            jax.ShapeDtypeStruct((b, _K, c), jnp.float32),
            jax.ShapeDtypeStruct((b, n, _K), jnp.float32),
        ],
    )(x, p0)

    return cluster3.reshape(b, n), proto, sim


# speedup vs baseline: 1.3752x; 1.3752x over previous
import jax
import jax.numpy as jnp
from jax import lax
from jax.experimental import pallas as pl
from jax.experimental.pallas import tpu as pltpu

_K = 128
_STAGES = 10


def _dfc_body(x_ref, p0_ref, cluster_ref, proto_ref, sim_ref):
    xc = x_ref[0]
    xt = jnp.transpose(xc)
    n = xc.shape[1]

    def stage(_, carry):
        pT, _ = carry
        simT = jnp.dot(pT, xc, preferred_element_type=jnp.float32)
        m = jnp.max(simT, axis=0, keepdims=True)
        e = jnp.exp(simT - m)
        sT = e * (1.0 / jnp.sum(e, axis=0, keepdims=True))
        q = jnp.dot(sT, xt, preferred_element_type=jnp.float32)
        rs = jnp.sum(sT, axis=1, keepdims=True)
        pTn = pT * 0.5 + q * (0.5 / rs)
        return pTn, sT

    pT, sT = lax.fori_loop(
        0, _STAGES, stage,
        (p0_ref[0], jnp.zeros((_K, n), jnp.float32)))

    proto_ref[0] = pT
    sim_ref[0] = jnp.transpose(sT)
    m = jnp.max(sT, axis=0, keepdims=True)
    idx = lax.broadcasted_iota(jnp.int32, (_K, n), 0)
    cluster_ref[0, 0] = jnp.min(jnp.where(sT == m, idx, _K), axis=0)


def kernel(x):
    b, c, n = x.shape
    p0T = jnp.transpose(x[:, :, :: n // _K], (0, 2, 1))

    cluster3, proto, sim = pl.pallas_call(
        _dfc_body,
        grid=(b,),
        in_specs=[
            pl.BlockSpec((1, c, n), lambda i: (i, 0, 0)),
            pl.BlockSpec((1, _K, c), lambda i: (i, 0, 0)),
        ],
        out_specs=[
            pl.BlockSpec((1, 1, n), lambda i: (i, 0, 0)),
            pl.BlockSpec((1, _K, c), lambda i: (i, 0, 0)),
            pl.BlockSpec((1, n, _K), lambda i: (i, 0, 0)),
        ],
        out_shape=[
            jax.ShapeDtypeStruct((b, 1, n), jnp.int32),
            jax.ShapeDtypeStruct((b, _K, c), jnp.float32),
            jax.ShapeDtypeStruct((b, n, _K), jnp.float32),
        ],
    )(x, p0T)

    return cluster3.reshape(b, n), proto, sim
